# baseline (speedup 1.0000x reference)
import jax
import jax.numpy as jnp
from jax import lax
from jax.experimental import pallas as pl
from jax.experimental.pallas import tpu as pltpu
from jax.experimental.pallas import tpu_sc as plsc

BATCH = 16384
EMBED_DIM = 32
NROWS = 1000001
LANES = 16
CHUNK = 128
RATING_SCALE = 5.5

_info = plsc.get_sparse_core_info()
_NC, _NS = _info.num_cores, _info.num_subcores
NW = _NC * _NS
BPW = BATCH // NW
NCHUNK = BPW // CHUNK
NGROUP = BPW // LANES
NVEC = CHUNK // LANES


def _ncf_body(users_hbm, items_hbm, wuf_hbm, wif_hbm, bu_hbm, bi_hbm,
              out_hbm,
              u_idx, i_idx, u_shift, i_shift, u_cols, i_cols,
              u_b, i_b, out_v, sem, bsem):
    wid = lax.axis_index("s") * _NC + lax.axis_index("c")
    base = wid * BPW

    pltpu.sync_copy(users_hbm.at[pl.ds(wid * NCHUNK, NCHUNK)], u_idx)
    pltpu.sync_copy(items_hbm.at[pl.ds(wid * NCHUNK, NCHUNK)], i_idx)

    bias_handles = []
    for j in range(NCHUNK):
        sl = pl.ds(j * CHUNK, CHUNK)
        bias_handles.append(pltpu.async_copy(bu_hbm.at[u_idx.at[j]], u_b.at[sl], bsem))
        bias_handles.append(pltpu.async_copy(bi_hbm.at[i_idx.at[j]], i_b.at[sl], bsem))

    def fetch(d, carry):
        for j in range(NCHUNK):
            for v in range(NVEC):
                sl = pl.ds(v * LANES, LANES)
                u_shift[d, j, sl] = u_idx[j, sl] + d * NROWS
                i_shift[d, j, sl] = i_idx[j, sl] + d * NROWS
        for j in range(NCHUNK):
            sl = pl.ds(d * BPW + j * CHUNK, CHUNK)
            pltpu.async_copy(wuf_hbm.at[u_shift.at[d].at[j]], u_cols.at[sl], sem)
            pltpu.async_copy(wif_hbm.at[i_shift.at[d].at[j]], i_cols.at[sl], sem)
        return carry

    lax.fori_loop(0, EMBED_DIM, fetch, 0, unroll=False)

    pltpu.make_async_copy(wuf_hbm.at[pl.ds(0, EMBED_DIM * BPW)], u_cols, sem).wait()
    pltpu.make_async_copy(wif_hbm.at[pl.ds(0, EMBED_DIM * BPW)], i_cols, sem).wait()

    for h in bias_handles:
        h.wait()

    def group(g, carry):
        sl = pl.ds(g * LANES, LANES)
        acc = u_b[sl] + i_b[sl]
        for d in range(EMBED_DIM):
            dsl = pl.ds(d * BPW + g * LANES, LANES)
            acc = acc + u_cols[dsl] * i_cols[dsl]
        out_v[sl] = RATING_SCALE / (1.0 + jnp.exp(-acc))
        return carry

    lax.fori_loop(0, NGROUP, group, 0, unroll=False)

    pltpu.sync_copy(out_v, out_hbm.at[pl.ds(base, BPW)])


def kernel(users, items, W_user, W_item, B_user, B_item):
    u = users.reshape(BATCH // CHUNK, CHUNK).astype(jnp.int32)
    it = items.reshape(BATCH // CHUNK, CHUNK).astype(jnp.int32)
    mesh = plsc.VectorSubcoreMesh(core_axis_name="c", subcore_axis_name="s")
    f = pl.kernel(
        _ncf_body,
        out_type=jax.ShapeDtypeStruct((BATCH,), jnp.float32),
        mesh=mesh,
        compiler_params=pltpu.CompilerParams(
            needs_layout_passes=False, use_tc_tiling_on_sc=False),
        scratch_types=[
            pltpu.VMEM((NCHUNK, CHUNK), jnp.int32),
            pltpu.VMEM((NCHUNK, CHUNK), jnp.int32),
            pltpu.VMEM((EMBED_DIM, NCHUNK, CHUNK), jnp.int32),
            pltpu.VMEM((EMBED_DIM, NCHUNK, CHUNK), jnp.int32),
            pltpu.VMEM((EMBED_DIM * BPW,), jnp.float32),
            pltpu.VMEM((EMBED_DIM * BPW,), jnp.float32),
            pltpu.VMEM((BPW,), jnp.float32),
            pltpu.VMEM((BPW,), jnp.float32),
            pltpu.VMEM((BPW,), jnp.float32),
            pltpu.SemaphoreType.DMA,
            pltpu.SemaphoreType.DMA,
        ],
    )
    return f(u, it, W_user.T.reshape(-1), W_item.T.reshape(-1),
             B_user.reshape(-1), B_item.reshape(-1))
